# baseline (speedup 1.0000x reference)
import functools

import jax
import jax.numpy as jnp
from jax import lax
from jax.experimental import pallas as pl
from jax.experimental.pallas import tpu as pltpu
from jax.experimental.pallas import tpu_sc as plsc

NUM_CORES = 2
NUM_SUBCORES = 16
NUM_WORKERS = NUM_CORES * NUM_SUBCORES

NB_ROWS = 4096
NB_COLS = 200
B = NB_ROWS * NB_COLS
D = 25
DP = 32
VOCAB = 100000
G = 128
ST = NB_COLS // 8
TILES = ST * (NB_ROWS // G)
TILE_IDX = 8 * G


@functools.partial(
    pl.kernel,
    out_type=jax.ShapeDtypeStruct((D, TILES, 8, G), jnp.float32),
    mesh=plsc.VectorSubcoreMesh(
        core_axis_name="c", subcore_axis_name="s",
        num_cores=NUM_CORES, num_subcores=NUM_SUBCORES),
    scratch_types=[
        pltpu.VMEM((TILE_IDX,), jnp.int32),
        pltpu.VMEM((TILE_IDX,), jnp.int32),
        pltpu.VMEM((TILE_IDX, DP), jnp.float32),
        pltpu.VMEM((TILE_IDX, DP), jnp.float32),
        pltpu.VMEM((D, 1, 8, G), jnp.float32),
        pltpu.VMEM((D, 1, 8, G), jnp.float32),
        pltpu.SemaphoreType.DMA,
        pltpu.SemaphoreType.DMA,
        pltpu.SemaphoreType.DMA,
        pltpu.SemaphoreType.DMA,
        pltpu.SemaphoreType.DMA,
        pltpu.SemaphoreType.DMA,
    ],
    compiler_params=pltpu.CompilerParams(
        use_tc_tiling_on_sc=False, needs_layout_passes=False),
)
def _gather_kernel(table_hbm, idx_hbm, out_hbm,
                   idx0, idx1, rows0, rows1, tile0, tile1,
                   sem_i0, sem_i1, sem_g0, sem_g1, sem_o0, sem_o1):
    wid = lax.axis_index("s") * NUM_CORES + lax.axis_index("c")
    col_base = wid * G

    iota = lax.iota(jnp.int32, 16)

    def fire_idx(st, idx_v, sem):
        return [
            pltpu.async_copy(
                idx_hbm.at[pl.ds((st * 8 + j) * NB_ROWS + col_base, G)],
                idx_v.at[pl.ds(j * G, G)],
                sem,
            )
            for j in range(8)
        ]

    def wait_idx(idx_v, sem):
        for j in range(8):
            pltpu.make_async_copy(
                idx_hbm.at[pl.ds(0, G)], idx_v.at[pl.ds(j * G, G)], sem
            ).wait()

    def fire_gathers(idx_v, rows_v, sem):
        return [
            pltpu.async_copy(
                table_hbm.at[idx_v.at[pl.ds(j * G, G)]],
                rows_v.at[pl.ds(j * G, G)],
                sem,
            )
            for j in range(8)
        ]

    def transpose_tile(rows_v, tile_v):
        def tbody(t, _):
            c = t // 8
            j = t % 8
            base = j * G
            cvec = iota * 0 + c
            for b0 in range(0, G, 16):
                v = plsc.load_gather(rows_v, [base + b0 + iota, cvec])
                tile_v[c, 0, j, pl.ds(b0, 16)] = v
            return ()

        lax.fori_loop(0, D * 8, tbody, (), unroll=2)

    def fire_out(st, tile_v, sem):
        return pltpu.async_copy(
            tile_v, out_hbm.at[:, pl.ds(st * (NB_ROWS // G) + wid, 1)], sem)

    def drain_out(tile_v, sem):
        pltpu.make_async_copy(
            tile_v, out_hbm.at[:, pl.ds(0, 1)], sem).wait()

    def process(st, idx_v, rows_v, tile_v, sem_i, sem_ip, idx_p,
                sem_g, sem_o):
        wait_idx(idx_v, sem_i)
        gs = fire_gathers(idx_v, rows_v, sem_g)

        @pl.when(st < ST - 1)
        def _():
            fire_idx(st + 1, idx_p, sem_ip)

        for cp in gs:
            cp.wait()

        @pl.when(st >= 2)
        def _():
            drain_out(tile_v, sem_o)

        transpose_tile(rows_v, tile_v)
        fire_out(st, tile_v, sem_o)

    fire_idx(0, idx0, sem_i0)

    def tile_body(st, _):
        @pl.when(st % 2 == 0)
        def _():
            process(st, idx0, rows0, tile0, sem_i0, sem_i1, idx1,
                    sem_g0, sem_o0)

        @pl.when(st % 2 == 1)
        def _():
            process(st, idx1, rows1, tile1, sem_i1, sem_i0, idx0,
                    sem_g1, sem_o1)
        return ()

    lax.fori_loop(0, ST, tile_body, ())

    drain_out(tile1, sem_o1)
    drain_out(tile0, sem_o0)


def kernel(x, table):
    idx = x.T.reshape(-1).astype(jnp.int32)
    table_p = jnp.pad(table, ((0, 0), (0, DP - D)))
    out = _gather_kernel(table_p, idx)
    r5 = out.reshape(D, ST, NB_ROWS // G, 8, G)
    return r5.transpose(2, 4, 1, 3, 0).reshape(NB_ROWS, NB_COLS, D)


# speedup vs baseline: 2.3762x; 2.3762x over previous
import functools

import jax
import jax.numpy as jnp
from jax import lax
from jax.experimental import pallas as pl
from jax.experimental.pallas import tpu as pltpu
from jax.experimental.pallas import tpu_sc as plsc

NUM_CORES = 2
NUM_SUBCORES = 16
NUM_WORKERS = NUM_CORES * NUM_SUBCORES

B = 4096 * 200
D = 25
DP = 32
VOCAB = 100000
G = 128
K = 10
BATCH = K * G
GROUPS = B // G
G_PER_W = GROUPS // NUM_WORKERS
NB = G_PER_W // K
NI = NB // 2


@functools.partial(
    pl.kernel,
    out_type=jax.ShapeDtypeStruct((B, 128), jnp.float32),
    mesh=plsc.VectorSubcoreMesh(
        core_axis_name="c", subcore_axis_name="s",
        num_cores=NUM_CORES, num_subcores=NUM_SUBCORES),
    scratch_types=[
        pltpu.VMEM((BATCH,), jnp.int32),
        pltpu.VMEM((BATCH,), jnp.int32),
        pltpu.VMEM((BATCH, DP), jnp.float32),
        pltpu.VMEM((BATCH, DP), jnp.float32),
        pltpu.SemaphoreType.DMA,
        pltpu.SemaphoreType.DMA,
        pltpu.SemaphoreType.DMA,
        pltpu.SemaphoreType.DMA,
        pltpu.SemaphoreType.DMA,
        pltpu.SemaphoreType.DMA,
    ],
    compiler_params=pltpu.CompilerParams(use_tc_tiling_on_sc=False),
)
def _gather_kernel(table_hbm, idx_hbm, out_hbm,
                   idx0, idx1, rows0, rows1,
                   sem_i0, sem_i1, sem_g0, sem_g1, sem_o0, sem_o1):
    wid = lax.axis_index("s") * NUM_CORES + lax.axis_index("c")
    row_base = wid * G_PER_W * G

    def fire_idx(b, idx_v, sem):
        return pltpu.async_copy(
            idx_hbm.at[pl.ds(row_base + b * BATCH, BATCH)], idx_v, sem)

    def wait_idx(idx_v, sem):
        pltpu.make_async_copy(idx_hbm.at[pl.ds(0, BATCH)], idx_v, sem).wait()

    def fire_gathers(idx_v, rows_v, sem):
        return [
            pltpu.async_copy(
                table_hbm.at[idx_v.at[pl.ds(j * G, G)]],
                rows_v.at[pl.ds(j * G, G)],
                sem,
            )
            for j in range(K)
        ]

    def fire_out(b, rows_v, sem):
        return pltpu.async_copy(
            rows_v,
            out_hbm.at[pl.ds(row_base + b * BATCH, BATCH), pl.ds(0, DP)],
            sem)

    def drain_out(rows_v, sem):
        pltpu.make_async_copy(
            rows_v, out_hbm.at[pl.ds(0, BATCH), pl.ds(0, DP)], sem).wait()

    fire_idx(0, idx0, sem_i0)
    fire_idx(1, idx1, sem_i1)

    def body(i, _):
        b0 = 2 * i
        b1 = b0 + 1

        @pl.when(i > 0)
        def _():
            drain_out(rows0, sem_o0)
        wait_idx(idx0, sem_i0)
        g0 = fire_gathers(idx0, rows0, sem_g0)

        @pl.when(i > 0)
        def _():
            drain_out(rows1, sem_o1)
        wait_idx(idx1, sem_i1)
        g1 = fire_gathers(idx1, rows1, sem_g1)

        for cp in g0:
            cp.wait()
        fire_out(b0, rows0, sem_o0)

        @pl.when(i < NI - 1)
        def _():
            fire_idx(b0 + 2, idx0, sem_i0)

        for cp in g1:
            cp.wait()
        fire_out(b1, rows1, sem_o1)

        @pl.when(i < NI - 1)
        def _():
            fire_idx(b1 + 2, idx1, sem_i1)
        return ()

    lax.fori_loop(0, NI, body, ())

    drain_out(rows0, sem_o0)
    drain_out(rows1, sem_o1)


def kernel(x, table):
    idx = x.T.reshape(-1).astype(jnp.int32)
    table_p = jnp.pad(table, ((0, 0), (0, DP - D)))
    out = _gather_kernel(table_p, idx)
    n_r, n_c = x.shape
    return out[:, :D].reshape(n_c, n_r, D).transpose(1, 0, 2)
